# baseline (speedup 1.0000x reference)
import functools

import jax
import jax.numpy as jnp
from jax import lax
from jax.experimental import pallas as pl
from jax.experimental.pallas import tpu as pltpu

_E = 64
_TOPK = 2
_TILE = 128


def _router_body(x_ref, rw_ref, idx_ref, w_ref):
    x = x_ref[...]
    rw = rw_ref[...]
    logits = lax.dot_general(
        x, rw, (((1,), (1,)), ((), ())), preferred_element_type=jnp.float32
    )
    e = logits.shape[1]
    iota = lax.broadcasted_iota(jnp.int32, logits.shape, 1)
    m1 = jnp.max(logits, axis=1, keepdims=True)
    a1 = jnp.min(jnp.where(logits == m1, iota, e), axis=1, keepdims=True)
    masked = jnp.where(iota == a1, -jnp.inf, logits)
    m2 = jnp.max(masked, axis=1, keepdims=True)
    a2 = jnp.min(jnp.where(masked == m2, iota, e), axis=1, keepdims=True)
    w1 = jax.nn.sigmoid(m1 - m2)
    idx_ref[...] = jnp.concatenate([a1, a2], axis=1)
    w_ref[...] = jnp.concatenate([w1, 1.0 - w1], axis=1)


def _route(flat, router_w):
    n, d = flat.shape
    bt = 1024
    grid = n // bt
    idxs, ws = pl.pallas_call(
        _router_body,
        grid=(grid,),
        in_specs=[
            pl.BlockSpec((bt, d), lambda i: (i, 0)),
            pl.BlockSpec((_E, d), lambda i: (0, 0)),
        ],
        out_specs=[
            pl.BlockSpec((bt, _TOPK), lambda i: (i, 0)),
            pl.BlockSpec((bt, _TOPK), lambda i: (i, 0)),
        ],
        out_shape=[
            jax.ShapeDtypeStruct((n, _TOPK), jnp.int32),
            jax.ShapeDtypeStruct((n, _TOPK), jnp.float32),
        ],
    )(flat, router_w)
    return idxs, ws


def _ffn_body(te_ref, act_ref, x_ref, gu_ref, dn_ref, w_ref, y_ref):
    del te_ref

    @pl.when(act_ref[pl.program_id(0)] > 0)
    def _():
        x = x_ref[...].astype(jnp.bfloat16)
        gu_w = gu_ref[0].astype(jnp.bfloat16)
        dn_w = dn_ref[0].astype(jnp.bfloat16)
        gu = lax.dot_general(
            x, gu_w, (((1,), (1,)), ((), ())), preferred_element_type=jnp.float32
        )
        dff = gu.shape[1] // 2
        gate = gu[:, :dff]
        up = gu[:, dff:]
        h = (gate * jax.nn.sigmoid(gate) * up).astype(jnp.bfloat16)
        y = lax.dot_general(
            h, dn_w, (((1,), (1,)), ((), ())), preferred_element_type=jnp.float32
        )
        y_ref[...] = y * w_ref[...]


def _grouped_ffn(x_sorted, gate_up_proj, down_proj, w_sorted, tile_expert, tile_active):
    cap, d = x_sorted.shape
    ntiles = cap // _TILE
    dff2 = gate_up_proj.shape[1]
    dff = down_proj.shape[2]
    grid_spec = pltpu.PrefetchScalarGridSpec(
        num_scalar_prefetch=2,
        grid=(ntiles,),
        in_specs=[
            pl.BlockSpec((_TILE, d), lambda t, te, act: (t, 0)),
            pl.BlockSpec((1, dff2, d), lambda t, te, act: (te[t], 0, 0)),
            pl.BlockSpec((1, d, dff), lambda t, te, act: (te[t], 0, 0)),
            pl.BlockSpec((_TILE, 1), lambda t, te, act: (t, 0)),
        ],
        out_specs=pl.BlockSpec((_TILE, d), lambda t, te, act: (t, 0)),
    )
    return pl.pallas_call(
        _ffn_body,
        grid_spec=grid_spec,
        out_shape=jax.ShapeDtypeStruct((cap, d), jnp.float32),
    )(tile_expert, tile_active, x_sorted, gate_up_proj, down_proj,
      w_sorted.reshape(cap, 1))


def kernel(hidden_states, gate_up_proj, down_proj, router_w):
    b, s, d = hidden_states.shape
    n = b * s
    npairs = n * _TOPK
    cap = npairs + _E * _TILE
    ntiles = cap // _TILE
    flat = hidden_states.reshape(n, d)

    idxs, ws = _route(flat, router_w)

    e_flat = idxs.reshape(-1)
    order = jnp.argsort(e_flat, stable=True)
    sorted_e = e_flat[order]
    counts = jnp.bincount(e_flat, length=_E)
    start = jnp.concatenate([jnp.zeros(1, jnp.int32), jnp.cumsum(counts)[:-1]])
    pad_counts = ((counts + _TILE - 1) // _TILE) * _TILE
    pad_cum = jnp.cumsum(pad_counts)
    pad_off = pad_cum - pad_counts
    total = pad_cum[-1]
    j = jnp.arange(npairs, dtype=jnp.int32)
    padpos = j + (pad_off - start)[sorted_e].astype(jnp.int32)
    src_token = jnp.zeros(cap, jnp.int32).at[padpos].set(
        (order // _TOPK).astype(jnp.int32))
    w_sorted = jnp.zeros(cap, jnp.float32).at[padpos].set(ws.reshape(-1)[order])
    pos_pair = jnp.zeros(npairs, jnp.int32).at[order].set(padpos)
    r0 = jnp.arange(ntiles, dtype=jnp.int32) * _TILE
    te_raw = jnp.searchsorted(pad_cum, r0, side="right").astype(jnp.int32)
    active = (r0 < total).astype(jnp.int32)
    last_tile = total // _TILE - 1
    e_last = te_raw[last_tile]
    tile_expert = jnp.where(active > 0, te_raw, e_last)

    x_sorted = flat[src_token]

    y = _grouped_ffn(x_sorted, gate_up_proj, down_proj, w_sorted,
                     tile_expert, active)

    pp = pos_pair.reshape(n, _TOPK)
    out = y[pp[:, 0]] + y[pp[:, 1]]
    return out.reshape(b, s, d)


# speedup vs baseline: 1.0612x; 1.0612x over previous
import functools

import jax
import jax.numpy as jnp
from jax import lax
from jax.experimental import pallas as pl
from jax.experimental.pallas import tpu as pltpu

_E = 64
_TOPK = 2
_TILE = 128


def _router_body(x_ref, rw_ref, idx_ref, w_ref):
    x = x_ref[...]
    rw = rw_ref[...]
    logits = lax.dot_general(
        x, rw, (((1,), (1,)), ((), ())), preferred_element_type=jnp.float32
    )
    e = logits.shape[1]
    iota = lax.broadcasted_iota(jnp.int32, logits.shape, 1)
    m1 = jnp.max(logits, axis=1, keepdims=True)
    a1 = jnp.min(jnp.where(logits == m1, iota, e), axis=1, keepdims=True)
    masked = jnp.where(iota == a1, -jnp.inf, logits)
    m2 = jnp.max(masked, axis=1, keepdims=True)
    a2 = jnp.min(jnp.where(masked == m2, iota, e), axis=1, keepdims=True)
    w1 = jax.nn.sigmoid(m1 - m2)
    idx_ref[...] = jnp.concatenate([a1, a2], axis=1)
    w_ref[...] = jnp.concatenate([w1, 1.0 - w1], axis=1)


def _route(flat, router_w):
    n, d = flat.shape
    bt = 1024
    grid = n // bt
    idxs, ws = pl.pallas_call(
        _router_body,
        grid=(grid,),
        in_specs=[
            pl.BlockSpec((bt, d), lambda i: (i, 0)),
            pl.BlockSpec((_E, d), lambda i: (0, 0)),
        ],
        out_specs=[
            pl.BlockSpec((bt, _TOPK), lambda i: (i, 0)),
            pl.BlockSpec((bt, _TOPK), lambda i: (i, 0)),
        ],
        out_shape=[
            jax.ShapeDtypeStruct((n, _TOPK), jnp.int32),
            jax.ShapeDtypeStruct((n, _TOPK), jnp.float32),
        ],
    )(flat, router_w)
    return idxs, ws


def _ffn_body(te_ref, act_ref, x_ref, gu_ref, dn_ref, w_ref, y_ref):
    del te_ref

    @pl.when(act_ref[pl.program_id(0)] > 0)
    def _():
        x = x_ref[...].astype(jnp.bfloat16)
        gu_w = gu_ref[0].astype(jnp.bfloat16)
        dn_w = dn_ref[0].astype(jnp.bfloat16)
        gu = lax.dot_general(
            x, gu_w, (((1,), (1,)), ((), ())), preferred_element_type=jnp.float32
        )
        dff = gu.shape[1] // 2
        gate = gu[:, :dff]
        up = gu[:, dff:]
        h = (gate * jax.nn.sigmoid(gate) * up).astype(jnp.bfloat16)
        y = lax.dot_general(
            h, dn_w, (((1,), (1,)), ((), ())), preferred_element_type=jnp.float32
        )
        y_ref[...] = y * w_ref[...]


def _grouped_ffn(x_sorted, gate_up_proj, down_proj, w_sorted, tile_expert, tile_active):
    cap, d = x_sorted.shape
    ntiles = cap // _TILE
    dff2 = gate_up_proj.shape[1]
    dff = down_proj.shape[2]
    grid_spec = pltpu.PrefetchScalarGridSpec(
        num_scalar_prefetch=2,
        grid=(ntiles,),
        in_specs=[
            pl.BlockSpec((_TILE, d), lambda t, te, act: (t, 0)),
            pl.BlockSpec((1, dff2, d), lambda t, te, act: (te[t], 0, 0)),
            pl.BlockSpec((1, d, dff), lambda t, te, act: (te[t], 0, 0)),
            pl.BlockSpec((_TILE, 1), lambda t, te, act: (t, 0)),
        ],
        out_specs=pl.BlockSpec((_TILE, d), lambda t, te, act: (t, 0)),
    )
    return pl.pallas_call(
        _ffn_body,
        grid_spec=grid_spec,
        out_shape=jax.ShapeDtypeStruct((cap, d), jnp.float32),
    )(tile_expert, tile_active, x_sorted, gate_up_proj, down_proj,
      w_sorted.reshape(cap, 1))


def kernel(hidden_states, gate_up_proj, down_proj, router_w):
    b, s, d = hidden_states.shape
    n = b * s
    npairs = n * _TOPK
    cap = npairs + _E * _TILE
    ntiles = cap // _TILE
    flat = hidden_states.reshape(n, d)

    idxs, ws = _route(flat, router_w)

    e_flat = idxs.reshape(-1)
    onehot = (e_flat[:, None] == jnp.arange(_E, dtype=jnp.int32)[None, :])
    cum = jnp.cumsum(onehot.astype(jnp.int32), axis=0)
    counts = cum[-1]
    rank = jnp.take_along_axis(cum, e_flat[:, None], axis=1)[:, 0] - 1
    pad_counts = ((counts + _TILE - 1) // _TILE) * _TILE
    pad_cum = jnp.cumsum(pad_counts)
    pad_off = pad_cum - pad_counts
    total = pad_cum[-1]
    pos_pair = pad_off[e_flat] + rank
    src_token = jnp.zeros(cap, jnp.int32).at[pos_pair].set(
        jnp.arange(npairs, dtype=jnp.int32) // _TOPK)
    w_sorted = jnp.zeros(cap, jnp.float32).at[pos_pair].set(ws.reshape(-1))
    r0 = jnp.arange(ntiles, dtype=jnp.int32) * _TILE
    te_raw = jnp.sum((r0[:, None] >= pad_cum[None, :]).astype(jnp.int32), axis=1)
    active = (r0 < total).astype(jnp.int32)
    last_tile = total // _TILE - 1
    e_last = te_raw[last_tile]
    tile_expert = jnp.where(active > 0, te_raw, e_last)

    x_sorted = flat[src_token]

    y = _grouped_ffn(x_sorted, gate_up_proj, down_proj, w_sorted,
                     tile_expert, active)

    pp = pos_pair.reshape(n, _TOPK)
    out = y[pp[:, 0]] + y[pp[:, 1]]
    return out.reshape(b, s, d)


# speedup vs baseline: 1.1887x; 1.1202x over previous
import functools

import jax
import jax.numpy as jnp
from jax import lax
from jax.experimental import pallas as pl
from jax.experimental.pallas import tpu as pltpu

_E = 64
_TOPK = 2
_TILE = 128


def _router_body(x_ref, rw_ref, idx_ref, w_ref):
    x = x_ref[...]
    rw = rw_ref[...]
    logits = lax.dot_general(
        x, rw, (((1,), (1,)), ((), ())), preferred_element_type=jnp.float32
    )
    e = logits.shape[1]
    iota = lax.broadcasted_iota(jnp.int32, logits.shape, 1)
    m1 = jnp.max(logits, axis=1, keepdims=True)
    a1 = jnp.min(jnp.where(logits == m1, iota, e), axis=1, keepdims=True)
    masked = jnp.where(iota == a1, -jnp.inf, logits)
    m2 = jnp.max(masked, axis=1, keepdims=True)
    a2 = jnp.min(jnp.where(masked == m2, iota, e), axis=1, keepdims=True)
    w1 = jax.nn.sigmoid(m1 - m2)
    idx_ref[...] = jnp.concatenate([a1, a2], axis=1)
    w_ref[...] = jnp.concatenate([w1, 1.0 - w1], axis=1)


def _route(flat, router_w):
    n, d = flat.shape
    bt = 1024
    grid = n // bt
    idxs, ws = pl.pallas_call(
        _router_body,
        grid=(grid,),
        in_specs=[
            pl.BlockSpec((bt, d), lambda i: (i, 0)),
            pl.BlockSpec((_E, d), lambda i: (0, 0)),
        ],
        out_specs=[
            pl.BlockSpec((bt, _TOPK), lambda i: (i, 0)),
            pl.BlockSpec((bt, _TOPK), lambda i: (i, 0)),
        ],
        out_shape=[
            jax.ShapeDtypeStruct((n, _TOPK), jnp.int32),
            jax.ShapeDtypeStruct((n, _TOPK), jnp.float32),
        ],
    )(flat, router_w)
    return idxs, ws


def _ffn_body(te_ref, sel_ref, act_ref, x_ref, gu_ref, dn_ref, y_ref):
    del te_ref, sel_ref

    @pl.when(act_ref[pl.program_id(0)] > 0)
    def _():
        x = x_ref[...].astype(jnp.bfloat16)
        gu_w = gu_ref[0].astype(jnp.bfloat16)
        dn_w = dn_ref[0].astype(jnp.bfloat16)
        gu = lax.dot_general(
            x, gu_w, (((1,), (1,)), ((), ())), preferred_element_type=jnp.float32
        )
        dff = gu.shape[1] // 2
        gate = gu[:, :dff]
        up = gu[:, dff:]
        h = (gate * jax.nn.sigmoid(gate) * up).astype(jnp.bfloat16)
        y = lax.dot_general(
            h, dn_w, (((1,), (1,)), ((), ())), preferred_element_type=jnp.float32
        )
        y_ref[...] = y


def _grouped_ffn(x_sorted, gate_up_proj, down_proj, tile_expert, tile_sel,
                 tile_active):
    cap, d = x_sorted.shape
    ntiles = cap // _TILE
    dff2 = gate_up_proj.shape[1]
    dff = down_proj.shape[2]
    grid_spec = pltpu.PrefetchScalarGridSpec(
        num_scalar_prefetch=3,
        grid=(ntiles,),
        in_specs=[
            pl.BlockSpec((_TILE, d), lambda t, te, sel, act: (sel[t], 0)),
            pl.BlockSpec((1, dff2, d), lambda t, te, sel, act: (te[t], 0, 0)),
            pl.BlockSpec((1, d, dff), lambda t, te, sel, act: (te[t], 0, 0)),
        ],
        out_specs=pl.BlockSpec((_TILE, d), lambda t, te, sel, act: (sel[t], 0)),
    )
    return pl.pallas_call(
        _ffn_body,
        grid_spec=grid_spec,
        out_shape=jax.ShapeDtypeStruct((cap, d), jnp.float32),
    )(tile_expert, tile_sel, tile_active, x_sorted, gate_up_proj, down_proj)


def kernel(hidden_states, gate_up_proj, down_proj, router_w):
    b, s, d = hidden_states.shape
    n = b * s
    npairs = n * _TOPK
    cap = npairs + _E * _TILE
    ntiles = cap // _TILE
    flat = hidden_states.reshape(n, d)

    idxs, ws = _route(flat, router_w)

    e_flat = idxs.reshape(-1)
    gl = 128
    ng = npairs // gl
    eg = e_flat.reshape(ng, gl)
    oh = (eg[:, :, None] == jnp.arange(_E, dtype=jnp.int32)[None, None, :]
          ).astype(jnp.float32)
    ri = jnp.arange(gl, dtype=jnp.int32)
    lt_incl = (ri[:, None] >= ri[None, :]).astype(jnp.float32)
    within = jnp.einsum("rl,gle->gre", lt_incl, oh,
                        preferred_element_type=jnp.float32)
    gsum = within[:, -1, :]
    gi = jnp.arange(ng, dtype=jnp.int32)
    lt_strict = (gi[:, None] > gi[None, :]).astype(jnp.float32)
    goff = jnp.einsum("hg,ge->he", lt_strict, gsum,
                      preferred_element_type=jnp.float32)
    counts = (goff[-1] + gsum[-1]).astype(jnp.int32)
    rank_f = (within + goff[:, None, :]).reshape(npairs, _E)
    rank = jnp.take_along_axis(rank_f, e_flat[:, None], axis=1)[:, 0]
    rank = rank.astype(jnp.int32) - 1
    pad_counts = ((counts + _TILE - 1) // _TILE) * _TILE
    pad_cum = jnp.cumsum(pad_counts)
    pad_off = pad_cum - pad_counts
    total = pad_cum[-1]
    pos_pair = pad_off[e_flat] + rank
    src_token = jnp.zeros(cap, jnp.int32).at[pos_pair].set(
        jnp.arange(npairs, dtype=jnp.int32) // _TOPK)
    r0 = jnp.arange(ntiles, dtype=jnp.int32) * _TILE
    te_raw = jnp.sum((r0[:, None] >= pad_cum[None, :]).astype(jnp.int32), axis=1)
    active = (r0 < total).astype(jnp.int32)
    last_tile = total // _TILE - 1
    e_last = te_raw[last_tile]
    tile_expert = jnp.where(active > 0, te_raw, e_last)
    tile_sel = jnp.where(active > 0, jnp.arange(ntiles, dtype=jnp.int32),
                         last_tile)

    x_sorted = flat[src_token]

    y = _grouped_ffn(x_sorted, gate_up_proj, down_proj, tile_expert, tile_sel,
                     active)

    pp = pos_pair.reshape(n, _TOPK)
    out = y[pp[:, 0]] * ws[:, :1] + y[pp[:, 1]] * ws[:, 1:]
    return out.reshape(b, s, d)


# speedup vs baseline: 1.1892x; 1.0004x over previous
import functools

import jax
import jax.numpy as jnp
from jax import lax
from jax.experimental import pallas as pl
from jax.experimental.pallas import tpu as pltpu
from jax.experimental.pallas import tpu_sc as plsc

_E = 64
_TOPK = 2
_TILE = 128


def _router_body(x_ref, rw_ref, idx_ref, w_ref):
    x = x_ref[...]
    rw = rw_ref[...]
    logits = lax.dot_general(
        x, rw, (((1,), (1,)), ((), ())), preferred_element_type=jnp.float32
    )
    e = logits.shape[1]
    iota = lax.broadcasted_iota(jnp.int32, logits.shape, 1)
    m1 = jnp.max(logits, axis=1, keepdims=True)
    a1 = jnp.min(jnp.where(logits == m1, iota, e), axis=1, keepdims=True)
    masked = jnp.where(iota == a1, -jnp.inf, logits)
    m2 = jnp.max(masked, axis=1, keepdims=True)
    a2 = jnp.min(jnp.where(masked == m2, iota, e), axis=1, keepdims=True)
    w1 = jax.nn.sigmoid(m1 - m2)
    idx_ref[...] = jnp.concatenate([a1, a2], axis=1)
    w_ref[...] = jnp.concatenate([w1, 1.0 - w1], axis=1)


def _route(flat, router_w):
    n, d = flat.shape
    bt = 1024
    grid = n // bt
    idxs, ws = pl.pallas_call(
        _router_body,
        grid=(grid,),
        in_specs=[
            pl.BlockSpec((bt, d), lambda i: (i, 0)),
            pl.BlockSpec((_E, d), lambda i: (0, 0)),
        ],
        out_specs=[
            pl.BlockSpec((bt, _TOPK), lambda i: (i, 0)),
            pl.BlockSpec((bt, _TOPK), lambda i: (i, 0)),
        ],
        out_shape=[
            jax.ShapeDtypeStruct((n, _TOPK), jnp.int32),
            jax.ShapeDtypeStruct((n, _TOPK), jnp.float32),
        ],
    )(flat, router_w)
    return idxs, ws


def _ffn_body(te_ref, sel_ref, act_ref, x_ref, gu_ref, dn_ref, y_ref):
    del te_ref, sel_ref

    @pl.when(act_ref[pl.program_id(0)] > 0)
    def _():
        x = x_ref[...].astype(jnp.bfloat16)
        gu_w = gu_ref[0].astype(jnp.bfloat16)
        dn_w = dn_ref[0].astype(jnp.bfloat16)
        gu = lax.dot_general(
            x, gu_w, (((1,), (1,)), ((), ())), preferred_element_type=jnp.float32
        )
        dff = gu.shape[1] // 2
        gate = gu[:, :dff]
        up = gu[:, dff:]
        h = (gate * jax.nn.sigmoid(gate) * up).astype(jnp.bfloat16)
        y = lax.dot_general(
            h, dn_w, (((1,), (1,)), ((), ())), preferred_element_type=jnp.float32
        )
        y_ref[...] = y


def _grouped_ffn(x_sorted, gate_up_proj, down_proj, tile_expert, tile_sel,
                 tile_active):
    cap, d = x_sorted.shape
    ntiles = cap // _TILE
    dff2 = gate_up_proj.shape[1]
    dff = down_proj.shape[2]
    grid_spec = pltpu.PrefetchScalarGridSpec(
        num_scalar_prefetch=3,
        grid=(ntiles,),
        in_specs=[
            pl.BlockSpec((_TILE, d), lambda t, te, sel, act: (sel[t], 0)),
            pl.BlockSpec((1, dff2, d), lambda t, te, sel, act: (te[t], 0, 0)),
            pl.BlockSpec((1, d, dff), lambda t, te, sel, act: (te[t], 0, 0)),
        ],
        out_specs=pl.BlockSpec((_TILE, d), lambda t, te, sel, act: (sel[t], 0)),
    )
    return pl.pallas_call(
        _ffn_body,
        grid_spec=grid_spec,
        out_shape=jax.ShapeDtypeStruct((cap, d), jnp.float32),
    )(tile_expert, tile_sel, tile_active, x_sorted, gate_up_proj, down_proj)


def _sc_combine(y, p0, p1, w0, w1):
    n = p0.shape[0]
    d = y.shape[1]
    info = plsc.get_sparse_core_info()
    nw = info.num_cores * info.num_subcores
    per_w = n // nw
    chunk = 16
    nch = per_w // chunk
    mesh = plsc.VectorSubcoreMesh(core_axis_name="c", subcore_axis_name="s")

    @functools.partial(
        pl.kernel,
        mesh=mesh,
        out_type=jax.ShapeDtypeStruct((n, d), jnp.float32),
        scratch_types=[
            pltpu.VMEM((per_w,), jnp.int32),
            pltpu.VMEM((per_w,), jnp.int32),
            pltpu.VMEM((per_w, 16), jnp.float32),
            pltpu.VMEM((per_w, 16), jnp.float32),
            pltpu.VMEM((chunk, d), jnp.float32),
            pltpu.VMEM((chunk, d), jnp.float32),
            pltpu.SemaphoreType.DMA,
            pltpu.SemaphoreType.DMA,
        ],
    )
    def k(y_hbm, p0_hbm, p1_hbm, w0_hbm, w1_hbm, out_hbm,
          i0v, i1v, w0s, w1s, b0, b1, sem0, sem1):
        wid = lax.axis_index("s") * info.num_cores + lax.axis_index("c")
        base = wid * per_w
        pltpu.sync_copy(p0_hbm.at[pl.ds(base, per_w)], i0v)
        pltpu.sync_copy(p1_hbm.at[pl.ds(base, per_w)], i1v)
        pltpu.sync_copy(w0_hbm.at[pl.ds(base, per_w)], w0s)
        pltpu.sync_copy(w1_hbm.at[pl.ds(base, per_w)], w1s)

        def do_chunk(c, carry):
            cp0 = pltpu.async_copy(
                y_hbm.at[i0v.at[pl.ds(c * chunk, chunk)]], b0, sem0)
            cp1 = pltpu.async_copy(
                y_hbm.at[i1v.at[pl.ds(c * chunk, chunk)]], b1, sem1)
            cp0.wait()
            cp1.wait()

            def do_row(r, rc):
                a = w0s[c * chunk + r, :]
                bb = w1s[c * chunk + r, :]
                for kk in range(d // 16):
                    sl = pl.ds(kk * 16, 16)
                    b0[r, sl] = a * b0[r, sl] + bb * b1[r, sl]
                return rc

            lax.fori_loop(0, chunk, do_row, 0)
            pltpu.sync_copy(b0, out_hbm.at[pl.ds(base + c * chunk, chunk)])
            return carry

        lax.fori_loop(0, nch, do_chunk, 0)

    return k(y, p0, p1, w0, w1)


def kernel(hidden_states, gate_up_proj, down_proj, router_w):
    b, s, d = hidden_states.shape
    n = b * s
    npairs = n * _TOPK
    cap = npairs + _E * _TILE
    ntiles = cap // _TILE
    flat = hidden_states.reshape(n, d)

    idxs, ws = _route(flat, router_w)

    e_flat = idxs.reshape(-1)
    gl = 128
    ng = npairs // gl
    eg = e_flat.reshape(ng, gl)
    oh = (eg[:, :, None] == jnp.arange(_E, dtype=jnp.int32)[None, None, :]
          ).astype(jnp.float32)
    ri = jnp.arange(gl, dtype=jnp.int32)
    lt_incl = (ri[:, None] >= ri[None, :]).astype(jnp.float32)
    within = jnp.einsum("rl,gle->gre", lt_incl, oh,
                        preferred_element_type=jnp.float32)
    gsum = within[:, -1, :]
    gi = jnp.arange(ng, dtype=jnp.int32)
    lt_strict = (gi[:, None] > gi[None, :]).astype(jnp.float32)
    goff = jnp.einsum("hg,ge->he", lt_strict, gsum,
                      preferred_element_type=jnp.float32)
    counts = (goff[-1] + gsum[-1]).astype(jnp.int32)
    rank_f = (within + goff[:, None, :]).reshape(npairs, _E)
    rank = jnp.take_along_axis(rank_f, e_flat[:, None], axis=1)[:, 0]
    rank = rank.astype(jnp.int32) - 1
    pad_counts = ((counts + _TILE - 1) // _TILE) * _TILE
    pad_cum = jnp.cumsum(pad_counts)
    pad_off = pad_cum - pad_counts
    total = pad_cum[-1]
    pos_pair = pad_off[e_flat] + rank
    src_token = jnp.zeros(cap, jnp.int32).at[pos_pair].set(
        jnp.arange(npairs, dtype=jnp.int32) // _TOPK)
    r0 = jnp.arange(ntiles, dtype=jnp.int32) * _TILE
    te_raw = jnp.sum((r0[:, None] >= pad_cum[None, :]).astype(jnp.int32), axis=1)
    active = (r0 < total).astype(jnp.int32)
    last_tile = total // _TILE - 1
    e_last = te_raw[last_tile]
    tile_expert = jnp.where(active > 0, te_raw, e_last)
    tile_sel = jnp.where(active > 0, jnp.arange(ntiles, dtype=jnp.int32),
                         last_tile)

    x_sorted = flat[src_token]

    y = _grouped_ffn(x_sorted, gate_up_proj, down_proj, tile_expert, tile_sel,
                     active)

    pp = pos_pair.reshape(n, _TOPK)
    w0b = jnp.broadcast_to(ws[:, 0:1], (n, 16))
    w1b = jnp.broadcast_to(ws[:, 1:2], (n, 16))
    out = _sc_combine(y, pp[:, 0], pp[:, 1], w0b, w1b)
    return out.reshape(b, s, d)
